# Initial kernel scaffold; baseline (speedup 1.0000x reference)
import functools

import jax
import jax.numpy as jnp
from jax.experimental import pallas as pl
from jax.experimental.pallas import tpu as pltpu

_PERCENT_ON = 0.1
_INT_MIN = jnp.int32(-(2 ** 31))


def _kwinners_body(k, x_ref, duty_ref, bs_ref, out_ref, key_ref):
    n = x_ref.shape[1] * x_ref.shape[2]
    target_density = float(k) / float(n)
    bs = bs_ref[0, 0]
    factors = jnp.exp((target_density - duty_ref[...]) * bs)
    xb = x_ref[0]
    boosted = xb * factors
    bits = jax.lax.bitcast_convert_type(boosted, jnp.int32)
    ikey = bits ^ (jax.lax.shift_right_arithmetic(bits, 31)
                   & jnp.int32(0x7FFFFFFF))
    key_ref[...] = ikey

    def step(i, t):
        cand = t + jax.lax.shift_left(jnp.int32(1), jnp.int32(31) - i)
        cnt = jnp.sum((key_ref[...] >= cand).astype(jnp.int32))
        return jax.lax.select(cnt >= k, cand, t)

    thresh = jax.lax.fori_loop(0, 32, step, _INT_MIN)
    out_ref[0] = jnp.where(key_ref[...] >= thresh, xb, jnp.float32(0.0))


@jax.jit
def kernel(x, duty_cycle, boost_strength):
    b, c, h, w = x.shape
    hw = h * w
    n = c * hw
    k = int(round(n * _PERCENT_ON))
    xr = x.reshape(b, c, hw)
    duty = duty_cycle.reshape(c, 1).astype(jnp.float32)
    bs = jnp.asarray(boost_strength, jnp.float32).reshape(1, 1)

    out = pl.pallas_call(
        functools.partial(_kwinners_body, k),
        grid=(b,),
        in_specs=[
            pl.BlockSpec((1, c, hw), lambda i: (i, 0, 0)),
            pl.BlockSpec((c, 1), lambda i: (0, 0)),
            pl.BlockSpec((1, 1), lambda i: (0, 0)),
        ],
        out_specs=pl.BlockSpec((1, c, hw), lambda i: (i, 0, 0)),
        out_shape=jax.ShapeDtypeStruct((b, c, hw), jnp.float32),
        scratch_shapes=[pltpu.VMEM((c, hw), jnp.int32)],
    )(xr, duty, bs)
    return out.reshape(b, c, h, w)


# speedup vs baseline: 35.2323x; 35.2323x over previous
import functools

import jax
import jax.numpy as jnp
import numpy as np
from jax.experimental import pallas as pl
from jax.experimental.pallas import tpu as pltpu

_PERCENT_ON = 0.1
_INT_MIN = np.int32(-(2 ** 31))


def _kwinners_body(k, x_ref, duty_ref, bs_ref, out_ref, key_ref):
    n = x_ref.shape[1] * x_ref.shape[2]
    target_density = float(k) / float(n)
    bs = bs_ref[0, 0]
    factors = jnp.exp((target_density - duty_ref[...]) * bs)
    xb = x_ref[0]
    boosted = xb * factors
    bits = jax.lax.bitcast_convert_type(boosted, jnp.int32)
    ikey = bits ^ (jax.lax.shift_right_arithmetic(bits, 31)
                   & np.int32(0x7FFFFFFF))
    key_ref[...] = ikey

    def step(i, t):
        cand = t + jax.lax.shift_left(np.int32(1), np.int32(31) - i)
        cnt = jnp.sum((key_ref[...] >= cand).astype(jnp.int32))
        return jax.lax.select(cnt >= k, cand, t)

    thresh = jax.lax.fori_loop(0, 32, step, _INT_MIN)
    out_ref[0] = jnp.where(key_ref[...] >= thresh, xb, jnp.float32(0.0))


@jax.jit
def kernel(x, duty_cycle, boost_strength):
    b, c, h, w = x.shape
    hw = h * w
    n = c * hw
    k = int(round(n * _PERCENT_ON))
    xr = x.reshape(b, c, hw)
    duty = duty_cycle.reshape(c, 1).astype(jnp.float32)
    bs = jnp.asarray(boost_strength, jnp.float32).reshape(1, 1)

    out = pl.pallas_call(
        functools.partial(_kwinners_body, k),
        grid=(b,),
        in_specs=[
            pl.BlockSpec((1, c, hw), lambda i: (i, 0, 0)),
            pl.BlockSpec((c, 1), lambda i: (0, 0)),
            pl.BlockSpec((1, 1), lambda i: (0, 0)),
        ],
        out_specs=pl.BlockSpec((1, c, hw), lambda i: (i, 0, 0)),
        out_shape=jax.ShapeDtypeStruct((b, c, hw), jnp.float32),
        scratch_shapes=[pltpu.VMEM((c, hw), jnp.int32)],
    )(xr, duty, bs)
    return out.reshape(b, c, h, w)
